# baseline (speedup 1.0000x reference)
import functools

import jax
import jax.numpy as jnp
from jax.experimental import pallas as pl
from jax.experimental.pallas import tpu as pltpu

_B, _T, _D = 4, 8192, 2048
_H1, _E = 256, 64
_CHUNK = 512
_K = _T // _CHUNK


def _reduce_body(x_ref, o_ref):
    k = pl.program_id(1)
    part = jnp.sum(x_ref[0], axis=0)

    @pl.when(k == 0)
    def _init():
        o_ref[0, 0, :] = part

    @pl.when(k > 0)
    def _acc():
        o_ref[0, 0, :] += part


def _gate_body(m_ref, w1_ref, b1_ref, w2_ref, b2_ref, w_ref, i_ref):
    xm = m_ref[...] * (1.0 / _T)
    h = jnp.maximum(
        jnp.dot(xm, w1_ref[...], preferred_element_type=jnp.float32)
        + b1_ref[...], 0.0)
    g = (jnp.dot(h, w2_ref[...], preferred_element_type=jnp.float32)
         + b2_ref[...])
    gmax = jnp.max(g, axis=-1, keepdims=True)
    e = jnp.exp(g - gmax)
    p = e / jnp.sum(e, axis=-1, keepdims=True)
    iota = jax.lax.broadcasted_iota(jnp.int32, p.shape, 1)
    v1 = jnp.max(p, axis=-1, keepdims=True)
    i1 = jnp.min(jnp.where(p == v1, iota, _E), axis=-1, keepdims=True)
    p2 = jnp.where(iota == i1, -jnp.inf, p)
    v2 = jnp.max(p2, axis=-1, keepdims=True)
    i2 = jnp.min(jnp.where(p2 == v2, iota, _E), axis=-1, keepdims=True)
    s = v1 + v2
    w_ref[...] = jnp.concatenate([v1 / s, v2 / s], axis=1)
    i_ref[...] = jnp.concatenate([i1, i2], axis=1)


@functools.partial(jax.jit, static_argnames=("interpret",))
def _run(x, W1, b1, W2, b2, interpret=False):
    sums = pl.pallas_call(
        _reduce_body,
        grid=(_B, _K),
        in_specs=[pl.BlockSpec((1, _CHUNK, _D), lambda b, k: (b, k, 0))],
        out_specs=pl.BlockSpec((1, 1, _D), lambda b, k: (b, 0, 0)),
        out_shape=jax.ShapeDtypeStruct((_B, 1, _D), jnp.float32),
        compiler_params=pltpu.CompilerParams(
            dimension_semantics=("parallel", "arbitrary")),
        interpret=interpret,
    )(x)
    w, idx = pl.pallas_call(
        _gate_body,
        out_shape=[
            jax.ShapeDtypeStruct((_B, 2), jnp.float32),
            jax.ShapeDtypeStruct((_B, 2), jnp.int32),
        ],
        interpret=interpret,
    )(sums.reshape(_B, _D), W1, b1.reshape(1, _H1), W2, b2.reshape(1, _E))
    return w, idx


def kernel(x, W1, b1, W2, b2):
    return _run(x, W1, b1, W2, b2)


# speedup vs baseline: 1.0145x; 1.0145x over previous
import functools

import jax
import jax.numpy as jnp
from jax.experimental import pallas as pl
from jax.experimental.pallas import tpu as pltpu

_B, _T, _D = 4, 8192, 2048
_H1, _E = 256, 64
_CHUNK = 512
_K = _T // _CHUNK


def _body(x_ref, w1_ref, b1_ref, w2_ref, b2_ref, w_ref, i_ref, acc_ref):
    b = pl.program_id(0)
    k = pl.program_id(1)

    @pl.when(jnp.logical_and(b == 0, k == 0))
    def _init():
        acc_ref[...] = jnp.zeros_like(acc_ref)

    part = jnp.sum(x_ref[0], axis=0, keepdims=True)
    row = jax.lax.broadcasted_iota(jnp.int32, (8, 1), 0)
    acc_ref[...] += jnp.where(row == b, part, 0.0)

    @pl.when(jnp.logical_and(b == _B - 1, k == _K - 1))
    def _gate():
        xm = acc_ref[0:_B, :] * (1.0 / _T)
        h = jnp.maximum(
            jnp.dot(xm, w1_ref[...], preferred_element_type=jnp.float32)
            + b1_ref[...], 0.0)
        g = (jnp.dot(h, w2_ref[...], preferred_element_type=jnp.float32)
             + b2_ref[...])
        gmax = jnp.max(g, axis=-1, keepdims=True)
        e = jnp.exp(g - gmax)
        p = e / jnp.sum(e, axis=-1, keepdims=True)
        iota = jax.lax.broadcasted_iota(jnp.int32, p.shape, 1)
        v1 = jnp.max(p, axis=-1, keepdims=True)
        i1 = jnp.min(jnp.where(p == v1, iota, _E), axis=-1, keepdims=True)
        p2 = jnp.where(iota == i1, -jnp.inf, p)
        v2 = jnp.max(p2, axis=-1, keepdims=True)
        i2 = jnp.min(jnp.where(p2 == v2, iota, _E), axis=-1, keepdims=True)
        s = v1 + v2
        w_ref[...] = jnp.concatenate([v1 / s, v2 / s], axis=1)
        i_ref[...] = jnp.concatenate([i1, i2], axis=1)


@functools.partial(jax.jit, static_argnames=("interpret",))
def _run(x, W1, b1, W2, b2, interpret=False):
    w, idx = pl.pallas_call(
        _body,
        grid=(_B, _K),
        in_specs=[
            pl.BlockSpec((1, _CHUNK, _D), lambda b, k: (b, k, 0)),
            pl.BlockSpec((_D, _H1), lambda b, k: (0, 0)),
            pl.BlockSpec((1, _H1), lambda b, k: (0, 0)),
            pl.BlockSpec((_H1, _E), lambda b, k: (0, 0)),
            pl.BlockSpec((1, _E), lambda b, k: (0, 0)),
        ],
        out_specs=[
            pl.BlockSpec((_B, 2), lambda b, k: (0, 0)),
            pl.BlockSpec((_B, 2), lambda b, k: (0, 0)),
        ],
        out_shape=[
            jax.ShapeDtypeStruct((_B, 2), jnp.float32),
            jax.ShapeDtypeStruct((_B, 2), jnp.int32),
        ],
        scratch_shapes=[pltpu.VMEM((8, _D), jnp.float32)],
        compiler_params=pltpu.CompilerParams(
            dimension_semantics=("arbitrary", "arbitrary")),
        interpret=interpret,
    )(x, W1, b1.reshape(1, _H1), W2, b2.reshape(1, _E))
    return w, idx


def kernel(x, W1, b1, W2, b2):
    return _run(x, W1, b1, W2, b2)


# speedup vs baseline: 1.1166x; 1.1006x over previous
import functools

import jax
import jax.numpy as jnp
from jax.experimental import pallas as pl
from jax.experimental.pallas import tpu as pltpu

_B, _T, _D = 4, 8192, 2048
_H1, _E = 256, 64
_CHUNK = 1024
_K = _T // _CHUNK


def _body(x_ref, w1_ref, b1_ref, w2_ref, b2_ref, w_ref, i_ref, acc_ref):
    b = pl.program_id(0)
    k = pl.program_id(1)

    @pl.when(jnp.logical_and(b == 0, k == 0))
    def _init():
        acc_ref[...] = jnp.zeros_like(acc_ref)

    part = jnp.sum(x_ref[0], axis=0, keepdims=True)
    row = jax.lax.broadcasted_iota(jnp.int32, (8, 1), 0)
    acc_ref[...] += jnp.where(row == b, part, 0.0)

    @pl.when(jnp.logical_and(b == _B - 1, k == _K - 1))
    def _gate():
        xm = acc_ref[0:_B, :] * (1.0 / _T)
        h = jnp.maximum(
            jnp.dot(xm, w1_ref[...], preferred_element_type=jnp.float32)
            + b1_ref[...], 0.0)
        g = (jnp.dot(h, w2_ref[...], preferred_element_type=jnp.float32)
             + b2_ref[...])
        gmax = jnp.max(g, axis=-1, keepdims=True)
        e = jnp.exp(g - gmax)
        p = e / jnp.sum(e, axis=-1, keepdims=True)
        iota = jax.lax.broadcasted_iota(jnp.int32, p.shape, 1)
        v1 = jnp.max(p, axis=-1, keepdims=True)
        i1 = jnp.min(jnp.where(p == v1, iota, _E), axis=-1, keepdims=True)
        p2 = jnp.where(iota == i1, -jnp.inf, p)
        v2 = jnp.max(p2, axis=-1, keepdims=True)
        i2 = jnp.min(jnp.where(p2 == v2, iota, _E), axis=-1, keepdims=True)
        s = v1 + v2
        w_ref[...] = jnp.concatenate([v1 / s, v2 / s], axis=1)
        i_ref[...] = jnp.concatenate([i1, i2], axis=1)


@functools.partial(jax.jit, static_argnames=("interpret",))
def _run(x, W1, b1, W2, b2, interpret=False):
    w, idx = pl.pallas_call(
        _body,
        grid=(_B, _K),
        in_specs=[
            pl.BlockSpec((1, _CHUNK, _D), lambda b, k: (b, k, 0)),
            pl.BlockSpec((_D, _H1), lambda b, k: (0, 0)),
            pl.BlockSpec((1, _H1), lambda b, k: (0, 0)),
            pl.BlockSpec((_H1, _E), lambda b, k: (0, 0)),
            pl.BlockSpec((1, _E), lambda b, k: (0, 0)),
        ],
        out_specs=[
            pl.BlockSpec((_B, 2), lambda b, k: (0, 0)),
            pl.BlockSpec((_B, 2), lambda b, k: (0, 0)),
        ],
        out_shape=[
            jax.ShapeDtypeStruct((_B, 2), jnp.float32),
            jax.ShapeDtypeStruct((_B, 2), jnp.int32),
        ],
        scratch_shapes=[pltpu.VMEM((8, _D), jnp.float32)],
        compiler_params=pltpu.CompilerParams(
            dimension_semantics=("arbitrary", "arbitrary")),
        interpret=interpret,
    )(x, W1, b1.reshape(1, _H1), W2, b2.reshape(1, _E))
    return w, idx


def kernel(x, W1, b1, W2, b2):
    return _run(x, W1, b1, W2, b2)
